# Initial kernel scaffold; baseline (speedup 1.0000x reference)
import jax
import jax.numpy as jnp
from jax.experimental import pallas as pl


def kernel(adj, embeds):
    raise NotImplementedError("write your pallas kernel here")


# speedup vs baseline: 6.0263x; 6.0263x over previous
import functools

import jax
import jax.numpy as jnp
from jax.experimental import pallas as pl
from jax.experimental.pallas import tpu as pltpu


def _round_up(x: int, m: int) -> int:
    return ((x + m - 1) // m) * m


def _gcn_rowblock_kernel(adj_ref, emb_ref, out_ref):
    out_ref[...] = jnp.dot(
        adj_ref[...].astype(jnp.bfloat16),
        emb_ref[...],
        preferred_element_type=jnp.float32,
    ).astype(out_ref.dtype)


@functools.partial(jax.jit, static_argnames=("bm",))
def _gcn(adj, embeds, bm: int = 512):
    M, K = adj.shape
    K2, D = embeds.shape
    out_dtype = embeds.dtype

    Dp = _round_up(D, 128)
    Mp = _round_up(M, bm)
    adj_p = adj if Mp == M else jnp.pad(adj, ((0, Mp - M), (0, 0)))
    emb_b = embeds.astype(jnp.bfloat16)
    if Dp != D:
        emb_b = jnp.pad(emb_b, ((0, 0), (0, Dp - D)))

    nm = Mp // bm
    cost = pl.CostEstimate(
        flops=2 * Mp * K * Dp,
        transcendentals=0,
        bytes_accessed=4 * M * K + 2 * K * Dp + 4 * Mp * Dp,
    )
    out = pl.pallas_call(
        _gcn_rowblock_kernel,
        out_shape=jax.ShapeDtypeStruct((Mp, Dp), out_dtype),
        grid=(nm,),
        in_specs=[
            pl.BlockSpec((bm, K), lambda i: (i, 0)),
            pl.BlockSpec((K, Dp), lambda i: (0, 0)),
        ],
        out_specs=pl.BlockSpec((bm, Dp), lambda i: (i, 0)),
        compiler_params=pltpu.CompilerParams(
            dimension_semantics=("parallel",)
        ),
        cost_estimate=cost,
    )(adj_p, emb_b)
    if Mp != M or Dp != D:
        out = out[:M, :D]
    return out


def kernel(adj, embeds):
    return _gcn(adj, embeds)


# speedup vs baseline: 6.7347x; 1.1176x over previous
import functools

import jax
import jax.numpy as jnp
from jax.experimental import pallas as pl
from jax.experimental.pallas import tpu as pltpu


def _round_up(x: int, m: int) -> int:
    return ((x + m - 1) // m) * m


def _gcn_rowblock_kernel(adj_ref, emb_ref, out_ref):
    out_ref[...] = jnp.dot(
        adj_ref[...].astype(jnp.bfloat16),
        emb_ref[...].astype(jnp.bfloat16),
        preferred_element_type=jnp.float32,
    ).astype(out_ref.dtype)


@functools.partial(jax.jit, static_argnames=("bm",))
def _gcn(adj, embeds, bm: int = 512):
    M, K = adj.shape
    K2, D = embeds.shape
    out_dtype = embeds.dtype

    Dp = _round_up(D, 128)
    Mp = _round_up(M, bm)
    adj_p = adj if Mp == M else jnp.pad(adj, ((0, Mp - M), (0, 0)))
    emb_b = embeds if Dp == D else jnp.pad(embeds, ((0, 0), (0, Dp - D)))

    nm = Mp // bm
    cost = pl.CostEstimate(
        flops=2 * Mp * K * Dp,
        transcendentals=0,
        bytes_accessed=4 * M * K + 4 * K * Dp + 4 * Mp * Dp,
    )
    out = pl.pallas_call(
        _gcn_rowblock_kernel,
        out_shape=jax.ShapeDtypeStruct((Mp, Dp), out_dtype),
        grid=(nm,),
        in_specs=[
            pl.BlockSpec((bm, K), lambda i: (i, 0)),
            pl.BlockSpec((K, Dp), lambda i: (0, 0)),
        ],
        out_specs=pl.BlockSpec((bm, Dp), lambda i: (i, 0)),
        compiler_params=pltpu.CompilerParams(
            dimension_semantics=("parallel",)
        ),
        cost_estimate=cost,
    )(adj_p, emb_b)
    if Mp != M or Dp != D:
        out = out[:M, :D]
    return out


def kernel(adj, embeds):
    return _gcn(adj, embeds)
